# baseline (speedup 1.0000x reference)
import functools

import jax
import jax.numpy as jnp
from jax import lax
from jax.experimental import pallas as pl
from jax.experimental.pallas import tpu as pltpu, tpu_sc as plsc

NC = 2
NS = 16
NW = NC * NS

MAXLEN = 200
EMBED = 64

CHUNK_SEQ = 2
CHUNK = CHUNK_SEQ * MAXLEN
SUB = 80
NSUB = CHUNK // SUB


def _tok_pos_kernel(nchunks, x_hbm, pos_hbm, tok_hbm, out_hbm,
                    idx_v, rows_v, pos_v, gsem):
    wid = lax.axis_index("s") * NC + lax.axis_index("c")

    for s in range(CHUNK_SEQ):
        pltpu.sync_copy(pos_hbm, pos_v.at[pl.ds(s * MAXLEN, MAXLEN)])

    def body(c, _):
        pltpu.sync_copy(x_hbm.at[wid, c], idx_v)
        copies = []
        for j in range(NSUB):
            copies.append(pltpu.async_copy(
                tok_hbm.at[idx_v.at[j]],
                rows_v.at[pl.ds(j * SUB, SUB)], gsem))
        for cp in copies:
            cp.wait()
        def add_row(r, _):
            for j in range(4):
                sl = pl.ds(j * 16, 16)
                rows_v[r, sl] += pos_v[r, sl]
            return ()
        lax.fori_loop(0, CHUNK, add_row, (), unroll=4)
        base = (wid * nchunks + c) * CHUNK
        pltpu.sync_copy(rows_v, out_hbm.at[pl.ds(base, CHUNK)])
        return ()

    lax.fori_loop(0, nchunks, body, ())


def kernel(x, tok_table, pos_table):
    B, L = x.shape
    V, E = tok_table.shape
    assert E == EMBED and L == MAXLEN
    total = B * L
    rows_per_w = total // NW
    nchunks = rows_per_w // CHUNK
    assert rows_per_w % CHUNK == 0

    x_resh = x.reshape(NW, nchunks, NSUB, SUB)

    fn = pl.kernel(
        functools.partial(_tok_pos_kernel, nchunks),
        out_type=jax.ShapeDtypeStruct((total, E), jnp.float32),
        mesh=plsc.VectorSubcoreMesh(core_axis_name="c", subcore_axis_name="s"),
        scratch_types=[
            pltpu.VMEM((NSUB, SUB), jnp.int32),
            pltpu.VMEM((CHUNK, E), jnp.float32),
            pltpu.VMEM((CHUNK, E), jnp.float32),
            pltpu.SemaphoreType.DMA,
        ],
        compiler_params=pltpu.CompilerParams(use_tc_tiling_on_sc=False),
    )
    out = fn(x_resh, pos_table, tok_table)
    return out.reshape(B, L, E)


# speedup vs baseline: 1.2494x; 1.2494x over previous
import functools

import jax
import jax.numpy as jnp
from jax import lax
from jax.experimental import pallas as pl
from jax.experimental.pallas import tpu as pltpu, tpu_sc as plsc

NC = 2
NS = 16
NW = NC * NS

MAXLEN = 200
EMBED = 64

CHUNK_SEQ = 2
SUB = 40
NSUB = MAXLEN // SUB


def _fire_gathers(tok_hbm, idx_v, buf, c, sem):
    copies = []
    for s in range(CHUNK_SEQ):
        for j in range(NSUB):
            copies.append(pltpu.async_copy(
                tok_hbm.at[idx_v.at[c * CHUNK_SEQ + s, pl.ds(j * SUB, SUB)]],
                buf.at[s, pl.ds(j * SUB, SUB)], sem))
    return copies


def _add_pos_and_flush(pos_v, buf, out_hbm, seq0, sem):
    def add_row(l, _):
        for j in range(EMBED // 16):
            sl = pl.ds(j * 16, 16)
            pvec = pos_v[l, sl]
            for s in range(CHUNK_SEQ):
                buf[s, l, sl] += pvec
        return ()
    lax.fori_loop(0, MAXLEN, add_row, (), unroll=2)
    pltpu.async_copy(buf, out_hbm.at[pl.ds(seq0, CHUNK_SEQ)], sem)


def _tok_pos_kernel(seq_per_w, x_hbm, pos_hbm, tok_hbm, out_hbm,
                    idx_v, rows0, rows1, pos_v,
                    gsem0, gsem1, osem0, osem1):
    wid = lax.axis_index("s") * NC + lax.axis_index("c")
    seq_base = wid * seq_per_w
    nchunks = seq_per_w // CHUNK_SEQ

    pltpu.sync_copy(x_hbm.at[pl.ds(seq_base, seq_per_w)], idx_v)
    pltpu.sync_copy(pos_hbm, pos_v)

    rows = (rows0, rows1)
    gsem = (gsem0, gsem1)
    osem = (osem0, osem1)

    def half(cc, par):
        c = cc * 2 + par
        buf, other = rows[par], rows[1 - par]

        @pl.when(cc >= 1)
        def _():
            pltpu.make_async_copy(
                buf, out_hbm.at[pl.ds(seq_base, CHUNK_SEQ)], osem[par]).wait()

        gathers = _fire_gathers(tok_hbm, idx_v, buf, c, gsem[par])

        @pl.when(c >= 1)
        def _():
            _add_pos_and_flush(pos_v, other, out_hbm,
                               seq_base + (c - 1) * CHUNK_SEQ, osem[1 - par])

        for cp in gathers:
            cp.wait()

    def body(cc, _):
        half(cc, 0)
        half(cc, 1)
        return ()

    lax.fori_loop(0, nchunks // 2, body, ())

    last = nchunks - 1
    _add_pos_and_flush(pos_v, rows[last % 2], out_hbm,
                       seq_base + last * CHUNK_SEQ, osem[last % 2])
    for par in range(2):
        pltpu.make_async_copy(
            rows[par], out_hbm.at[pl.ds(seq_base, CHUNK_SEQ)], osem[par]).wait()


def kernel(x, tok_table, pos_table):
    B, L = x.shape
    V, E = tok_table.shape
    assert E == EMBED and L == MAXLEN
    seq_per_w = B // NW
    assert B % NW == 0 and (seq_per_w // CHUNK_SEQ) % 2 == 0

    fn = pl.kernel(
        functools.partial(_tok_pos_kernel, seq_per_w),
        out_type=jax.ShapeDtypeStruct((B, L, E), jnp.float32),
        mesh=plsc.VectorSubcoreMesh(core_axis_name="c", subcore_axis_name="s"),
        scratch_types=[
            pltpu.VMEM((seq_per_w, L), jnp.int32),
            pltpu.VMEM((CHUNK_SEQ, L, E), jnp.float32),
            pltpu.VMEM((CHUNK_SEQ, L, E), jnp.float32),
            pltpu.VMEM((L, E), jnp.float32),
            pltpu.SemaphoreType.DMA,
            pltpu.SemaphoreType.DMA,
            pltpu.SemaphoreType.DMA,
            pltpu.SemaphoreType.DMA,
        ],
        compiler_params=pltpu.CompilerParams(use_tc_tiling_on_sc=False),
    )
    return fn(x, pos_table, tok_table)
